# baseline (speedup 1.0000x reference)
import functools
import math

import jax
import jax.numpy as jnp
from jax import lax
from jax.experimental import pallas as pl
from jax.experimental.pallas import tpu as pltpu
from jax.experimental.pallas import tpu_sc as plsc

N_EMBD = 64
SCALE = math.sqrt(N_EMBD)

NC = 2
NS = 16
NW = NC * NS
IDXW = 128
K = 8
CHUNK = K * IDXW
LANES = 16


@functools.partial(jax.jit, static_argnames=("chunks_per_w",))
def _emb_lookup(lut, idx3, chunks_per_w):
    n_chunks = idx3.shape[0]
    B = n_chunks * CHUNK
    mesh = plsc.VectorSubcoreMesh(core_axis_name="c", subcore_axis_name="s")

    @functools.partial(
        pl.kernel,
        mesh=mesh,
        compiler_params=pltpu.CompilerParams(use_tc_tiling_on_sc=False),
        out_type=jax.ShapeDtypeStruct((B, N_EMBD), jnp.float32),
        scratch_types=[
            pltpu.VMEM((K, IDXW), jnp.int32),
            pltpu.VMEM((CHUNK, N_EMBD), jnp.float32),
            pltpu.SemaphoreType.DMA,
        ],
    )
    def k(lut_hbm, idx_hbm, out_hbm, idx_v, rows_v, sem):
        wid = lax.axis_index("s") * NC + lax.axis_index("c")
        first = wid * chunks_per_w

        def chunk_body(g, carry):
            c = first + g
            pltpu.sync_copy(idx_hbm.at[c], idx_v)
            cps = [
                pltpu.async_copy(
                    lut_hbm.at[idx_v.at[j]],
                    rows_v.at[pl.ds(j * IDXW, IDXW)],
                    sem,
                )
                for j in range(K)
            ]
            for cp in cps:
                cp.wait()

            def scale_row(i, carry2):
                for j in range(N_EMBD // LANES):
                    sl = pl.ds(j * LANES, LANES)
                    rows_v[i, sl] = rows_v[i, sl] * SCALE
                return carry2

            lax.fori_loop(0, CHUNK, scale_row, 0)
            pltpu.sync_copy(rows_v, out_hbm.at[pl.ds(c * CHUNK, CHUNK)])
            return carry

        lax.fori_loop(0, chunks_per_w, chunk_body, 0)

    return k(lut, idx3)


def kernel(x, lut):
    B = x.shape[0] * x.shape[1]
    idx3 = x.reshape(-1, K, IDXW)
    chunks_per_w = idx3.shape[0] // NW
    out = _emb_lookup(lut, idx3, chunks_per_w)
    return out.reshape(x.shape[0], x.shape[1], N_EMBD)


# speedup vs baseline: 1.0471x; 1.0471x over previous
import functools
import math

import jax
import jax.numpy as jnp
from jax import lax
from jax.experimental import pallas as pl
from jax.experimental.pallas import tpu as pltpu
from jax.experimental.pallas import tpu_sc as plsc

N_EMBD = 64
SCALE = math.sqrt(N_EMBD)

NC = 2
NS = 16
NW = NC * NS
SEQ = 200
SPLITS = ((0, 104), (104, 96))
R = 4
LANES = 16


@jax.jit
def _emb_lookup(lut, x):
    n_rows = x.shape[0]
    rows_per_w = n_rows // NW
    n_chunks = rows_per_w // R
    mesh = plsc.VectorSubcoreMesh(core_axis_name="c", subcore_axis_name="s")

    @functools.partial(
        pl.kernel,
        mesh=mesh,
        compiler_params=pltpu.CompilerParams(use_tc_tiling_on_sc=False),
        out_type=jax.ShapeDtypeStruct((n_rows, SEQ, N_EMBD), jnp.float32),
        scratch_types=[
            pltpu.VMEM((rows_per_w, SEQ), jnp.int32),
            pltpu.VMEM((R, SEQ, N_EMBD), jnp.float32),
            pltpu.SemaphoreType.DMA,
        ],
    )
    def k(lut_hbm, x_hbm, out_hbm, idx_v, rows_v, sem):
        wid = lax.axis_index("s") * NC + lax.axis_index("c")
        row0 = wid * rows_per_w
        pltpu.sync_copy(x_hbm.at[pl.ds(row0, rows_per_w)], idx_v)

        def chunk_body(g, carry):
            cps = []
            for r in range(R):
                for off, width in SPLITS:
                    cps.append(
                        pltpu.async_copy(
                            lut_hbm.at[idx_v.at[g * R + r, pl.ds(off, width)]],
                            rows_v.at[r, pl.ds(off, width)],
                            sem,
                        )
                    )
            for cp in cps:
                cp.wait()

            def scale_pos(p, carry2):
                for r in range(R):
                    for j in range(N_EMBD // LANES):
                        sl = pl.ds(j * LANES, LANES)
                        rows_v[r, p, sl] = rows_v[r, p, sl] * SCALE
                return carry2

            lax.fori_loop(0, SEQ, scale_pos, 0)
            pltpu.sync_copy(rows_v, out_hbm.at[pl.ds(row0 + g * R, R)])
            return carry

        lax.fori_loop(0, n_chunks, chunk_body, 0)

    return k(lut, x)


def kernel(x, lut):
    return _emb_lookup(lut, x)
